# baseline (speedup 1.0000x reference)
import jax
import jax.numpy as jnp
from jax.experimental import pallas as pl
from jax.experimental.pallas import tpu as pltpu

EPS = 1e-16


def _cos_body(memt_ref, kbt_ref, k_ref, cos_ref, num_ref, ssq_ref):
    i1 = pl.program_id(1)
    kb = kbt_ref[...].T + EPS
    ns = memt_ref.shape[2]
    cb = 2048
    for c in range(ns // cb):
        sl = pl.ds(c * cb, cb)
        blk = memt_ref[:, :, sl]
        pnum = jnp.sum(blk * kb[:, :, None], axis=1)
        pssq = jnp.sum(blk * blk, axis=1)

        @pl.when(i1 == 0)
        def _():
            num_ref[:, sl] = pnum
            ssq_ref[:, sl] = pssq

        @pl.when(i1 > 0)
        def _():
            num_ref[:, sl] += pnum
            ssq_ref[:, sl] += pssq

    @pl.when(i1 == pl.num_programs(1) - 1)
    def _():
        kk = k_ref[...] + EPS
        normk = jnp.sqrt(jnp.sum(kk * kk, axis=-1))
        denom = jnp.sqrt(ssq_ref[...]) * normk[:, None]
        cos_ref[...] = num_ref[...] / jnp.maximum(denom, 1e-8)


def _w_body(cos_ref, wprev_ref, beta_ref, g_ref, s_ref, gamma_ref, w_ref):
    cos = cos_ref[...]
    beta = beta_ref[...]
    x = beta * cos
    x = x - jnp.max(x, axis=1, keepdims=True)
    ex = jnp.exp(x)
    wc = ex / jnp.sum(ex, axis=1, keepdims=True)
    g = g_ref[...]
    wg = g * wc + (1.0 - g) * wprev_ref[...]
    s = s_ref[...]
    left = jnp.concatenate([wg[:, -1:], wg[:, :-1]], axis=1)
    right = jnp.concatenate([wg[:, 1:], wg[:, :1]], axis=1)
    sh = left * s[:, 0:1] + wg * s[:, 1:2] + right * s[:, 2:3]
    gamma = gamma_ref[...]
    wpow = jnp.where(sh > 0.0,
                     jnp.exp(gamma * jnp.log(jnp.maximum(sh, 1e-38))),
                     0.0)
    w_ref[...] = wpow / (jnp.sum(wpow, axis=1, keepdims=True) + EPS)


def _read_body(w_ref, memt_ref, out_ref):
    @pl.when(pl.program_id(0) == 0)
    def _():
        out_ref[...] = jnp.zeros_like(out_ref)

    w = w_ref[...]
    memt = memt_ref[...]
    out_ref[...] += jnp.sum(memt * w[:, None, :], axis=2)


@jax.jit
def kernel(memory, k, beta, g, s, gamma, w_prev):
    B, N, M = memory.shape
    BN = min(2048, N)
    nb = N // BN
    memt = jnp.swapaxes(memory, 1, 2)

    MB = 8
    NS = N // 2
    cos = pl.pallas_call(
        _cos_body,
        grid=(N // NS, M // MB),
        in_specs=[
            pl.BlockSpec((B, MB, NS), lambda i0, i1: (0, i1, i0)),
            pl.BlockSpec((MB, B), lambda i0, i1: (i1, 0)),
            pl.BlockSpec((B, M), lambda i0, i1: (0, 0)),
        ],
        out_specs=pl.BlockSpec((B, NS), lambda i0, i1: (0, i0)),
        out_shape=jax.ShapeDtypeStruct((B, N), jnp.float32),
        scratch_shapes=[pltpu.VMEM((B, NS), jnp.float32),
                        pltpu.VMEM((B, NS), jnp.float32)],
    )(memt, k.T, k)

    w = pl.pallas_call(
        _w_body,
        in_specs=[pl.BlockSpec(x.shape, lambda: (0,) * x.ndim)
                  for x in (cos, w_prev, beta, g, s, gamma)],
        out_specs=pl.BlockSpec((B, N), lambda: (0, 0)),
        out_shape=jax.ShapeDtypeStruct((B, N), jnp.float32),
    )(cos, w_prev, beta, g, s, gamma)

    read = pl.pallas_call(
        _read_body,
        grid=(nb,),
        in_specs=[
            pl.BlockSpec((B, BN), lambda i: (0, i)),
            pl.BlockSpec((B, M, BN), lambda i: (0, 0, i)),
        ],
        out_specs=pl.BlockSpec((B, M), lambda i: (0, 0)),
        out_shape=jax.ShapeDtypeStruct((B, M), jnp.float32),
    )(w, memt)

    return read


# speedup vs baseline: 1.4895x; 1.4895x over previous
import jax
import jax.numpy as jnp
from jax.experimental import pallas as pl
from jax.experimental.pallas import tpu as pltpu

EPS = 1e-16


def _cos_body(memt_ref, k_ref, cos_ref):
    memt = memt_ref[...]
    kk = k_ref[...] + EPS
    num = jnp.sum(memt * kk[:, :, None], axis=1)
    ssq = jnp.sum(memt * memt, axis=1)
    normk = jnp.sqrt(jnp.sum(kk * kk, axis=-1))
    denom = jnp.sqrt(ssq) * normk[:, None]
    cos_ref[...] = num / jnp.maximum(denom, 1e-8)


def _w_body(cos_ref, wprev_ref, beta_ref, g_ref, s_ref, gamma_ref, w_ref):
    cos = cos_ref[...]
    beta = beta_ref[...]
    x = beta * cos
    x = x - jnp.max(x, axis=1, keepdims=True)
    ex = jnp.exp(x)
    wc = ex / jnp.sum(ex, axis=1, keepdims=True)
    g = g_ref[...]
    wg = g * wc + (1.0 - g) * wprev_ref[...]
    s = s_ref[...]
    left = jnp.concatenate([wg[:, -1:], wg[:, :-1]], axis=1)
    right = jnp.concatenate([wg[:, 1:], wg[:, :1]], axis=1)
    sh = left * s[:, 0:1] + wg * s[:, 1:2] + right * s[:, 2:3]
    gamma = gamma_ref[...]
    wpow = jnp.where(sh > 0.0,
                     jnp.exp(gamma * jnp.log(jnp.maximum(sh, 1e-38))),
                     0.0)
    w_ref[...] = wpow / (jnp.sum(wpow, axis=1, keepdims=True) + EPS)


def _read_body(cos_ref, wprev_ref, beta_ref, g_ref, s_ref, gamma_ref,
               memt_ref, out_ref, w_sref):
    i = pl.program_id(0)

    @pl.when(i == 0)
    def _():
        _w_body(cos_ref, wprev_ref, beta_ref, g_ref, s_ref, gamma_ref,
                w_sref)
        out_ref[...] = jnp.zeros_like(out_ref)

    bn = memt_ref.shape[2]
    w = w_sref[:, pl.ds(i * bn, bn)]
    memt = memt_ref[...]
    out_ref[...] += jnp.sum(memt * w[:, None, :], axis=2)


@jax.jit
def kernel(memory, k, beta, g, s, gamma, w_prev):
    B, N, M = memory.shape
    BN = min(2048, N)
    nb = N // BN
    memt = jnp.swapaxes(memory, 1, 2)

    cos = pl.pallas_call(
        _cos_body,
        grid=(nb,),
        in_specs=[
            pl.BlockSpec((B, M, BN), lambda i: (0, 0, i)),
            pl.BlockSpec((B, M), lambda i: (0, 0)),
        ],
        out_specs=pl.BlockSpec((B, BN), lambda i: (0, i)),
        out_shape=jax.ShapeDtypeStruct((B, N), jnp.float32),
    )(memt, k)

    read = pl.pallas_call(
        _read_body,
        grid=(nb,),
        in_specs=[pl.BlockSpec(x.shape, lambda i, _nd=x.ndim: (0,) * _nd)
                  for x in (cos, w_prev, beta, g, s, gamma)] + [
            pl.BlockSpec((B, M, BN), lambda i: (0, 0, i)),
        ],
        out_specs=pl.BlockSpec((B, M), lambda i: (0, 0)),
        out_shape=jax.ShapeDtypeStruct((B, M), jnp.float32),
        scratch_shapes=[pltpu.VMEM((B, N), jnp.float32)],
    )(cos, w_prev, beta, g, s, gamma, memt)

    return read
